# baseline (speedup 1.0000x reference)
import jax
import jax.numpy as jnp
from jax.experimental import pallas as pl
from jax.experimental.pallas import tpu as pltpu

_HW = [[68, 120], [34, 60], [17, 30]]
_STRIDES = [8.0, 16.0, 32.0]
_A0 = _HW[0][0] * _HW[0][1]
_A1 = _A0 + _HW[1][0] * _HW[1][1]
_A = _A1 + _HW[2][0] * _HW[2][1]
_C = 16
_K = _A * _C
_B = 64
_BBLK = 16


def _grid_xy(a_i32):
    in0 = a_i32 < _A0
    in1 = a_i32 < _A1
    stride = jnp.where(in0, _STRIDES[0], jnp.where(in1, _STRIDES[1], _STRIDES[2]))
    start = jnp.where(in0, 0.0, jnp.where(in1, float(_A0), float(_A1)))
    width = jnp.where(in0, float(_HW[0][1]), jnp.where(in1, float(_HW[1][1]),
                                                       float(_HW[2][1])))
    rel = a_i32.astype(jnp.float32) - start
    gy = jnp.floor(rel / width)
    gx = rel - gy * width
    return gx, gy, stride


def _decode_kernel(x_ref, chan_ref, gadd_ref, stride_ref, o_ref,
                   xs_ref, ys_ref, st_ref):
    chan = chan_ref[...]
    x = x_ref[...]
    decoded = jnp.where(chan < 2, x + gadd_ref[...], jnp.exp(x))
    o_ref[...] = jnp.where(chan < 4, decoded * stride_ref[...], x)

    @pl.when(pl.program_id(0) == 0)
    def _():
        ja = jax.lax.broadcasted_iota(jnp.int32, (1, _A), 1)
        agx, agy, astride = _grid_xy(ja)
        xs_ref[...] = agx
        ys_ref[...] = agy
        st_ref[...] = astride


def _col_consts():
    j = jax.lax.broadcasted_iota(jnp.int32, (1, _K), 1)
    a = j >> 4
    chan = j & 15
    gx, gy, stride = _grid_xy(a)
    gadd = jnp.where(chan == 0, gx, jnp.where(chan == 1, gy, 0.0))
    return chan, gadd, stride


@jax.jit
def _decode(x2):
    f32 = jnp.float32
    chan, gadd, stride = _col_consts()
    aux = pl.BlockSpec((1, _K), lambda i: (0, 0))
    out, xs, ys, st = pl.pallas_call(
        _decode_kernel,
        grid=(_B // _BBLK,),
        in_specs=[pl.BlockSpec((_BBLK, _K), lambda i: (i, 0)), aux, aux, aux],
        out_specs=[
            pl.BlockSpec((_BBLK, _K), lambda i: (i, 0)),
            pl.BlockSpec((1, _A), lambda i: (0, 0)),
            pl.BlockSpec((1, _A), lambda i: (0, 0)),
            pl.BlockSpec((1, _A), lambda i: (0, 0)),
        ],
        out_shape=[
            jax.ShapeDtypeStruct((_B, _K), f32),
            jax.ShapeDtypeStruct((1, _A), f32),
            jax.ShapeDtypeStruct((1, _A), f32),
            jax.ShapeDtypeStruct((1, _A), f32),
        ],
        compiler_params=pltpu.CompilerParams(
            allow_input_fusion=[True, True, True, True]),
    )(x2, chan, gadd, stride)
    return out, xs, ys, st


def kernel(outputs):
    x2 = outputs.reshape(_B, _K)
    out, xs, ys, st = _decode(x2)
    return out.reshape(_B, _A, _C), xs, ys, st


# speedup vs baseline: 1.0034x; 1.0034x over previous
import jax
import jax.numpy as jnp
from jax.experimental import pallas as pl
_HW = [[68, 120], [34, 60], [17, 30]]
_STRIDES = [8.0, 16.0, 32.0]
_A0 = _HW[0][0] * _HW[0][1]
_A1 = _A0 + _HW[1][0] * _HW[1][1]
_A = _A1 + _HW[2][0] * _HW[2][1]
_C = 16
_K = _A * _C
_B = 64
_BBLK = 8


def _grid_xy(a_i32):
    in0 = a_i32 < _A0
    in1 = a_i32 < _A1
    stride = jnp.where(in0, _STRIDES[0], jnp.where(in1, _STRIDES[1], _STRIDES[2]))
    start = jnp.where(in0, 0.0, jnp.where(in1, float(_A0), float(_A1)))
    width = jnp.where(in0, float(_HW[0][1]), jnp.where(in1, float(_HW[1][1]),
                                                       float(_HW[2][1])))
    rel = a_i32.astype(jnp.float32) - start
    gy = jnp.floor(rel / width)
    gx = rel - gy * width
    return gx, gy, stride


def _decode_kernel(x_ref, chan_ref, gadd_ref, stride_ref, o_ref,
                   xs_ref, ys_ref, st_ref):
    chan = chan_ref[...]
    x = x_ref[...]
    decoded = jnp.where(chan < 2, x + gadd_ref[...], jnp.exp(x))
    o_ref[...] = jnp.where(chan < 4, decoded * stride_ref[...], x)

    @pl.when(pl.program_id(0) == 0)
    def _():
        ja = jax.lax.broadcasted_iota(jnp.int32, (1, _A), 1)
        agx, agy, astride = _grid_xy(ja)
        xs_ref[...] = agx
        ys_ref[...] = agy
        st_ref[...] = astride


def _col_consts():
    j = jax.lax.broadcasted_iota(jnp.int32, (1, _K), 1)
    a = j >> 4
    chan = j & 15
    gx, gy, stride = _grid_xy(a)
    gadd = jnp.where(chan == 0, gx, jnp.where(chan == 1, gy, 0.0))
    return chan, gadd, stride


@jax.jit
def _decode(x2):
    f32 = jnp.float32
    chan, gadd, stride = _col_consts()
    aux = pl.BlockSpec((1, _K), lambda i: (0, 0))
    out, xs, ys, st = pl.pallas_call(
        _decode_kernel,
        grid=(_B // _BBLK,),
        in_specs=[pl.BlockSpec((_BBLK, _K), lambda i: (i, 0)), aux, aux, aux],
        out_specs=[
            pl.BlockSpec((_BBLK, _K), lambda i: (i, 0)),
            pl.BlockSpec((1, _A), lambda i: (0, 0)),
            pl.BlockSpec((1, _A), lambda i: (0, 0)),
            pl.BlockSpec((1, _A), lambda i: (0, 0)),
        ],
        out_shape=[
            jax.ShapeDtypeStruct((_B, _K), f32),
            jax.ShapeDtypeStruct((1, _A), f32),
            jax.ShapeDtypeStruct((1, _A), f32),
            jax.ShapeDtypeStruct((1, _A), f32),
        ],
    )(x2, chan, gadd, stride)
    return out, xs, ys, st


def kernel(outputs):
    x2 = outputs.reshape(_B, _K)
    out, xs, ys, st = _decode(x2)
    return out.reshape(_B, _A, _C), xs, ys, st
